# baseline (speedup 1.0000x reference)
import functools

import jax
import jax.numpy as jnp
from jax import lax
from jax.experimental import pallas as pl
from jax.experimental.pallas import tpu as pltpu
from jax.experimental.pallas import tpu_sc as plsc

VOCAB = 100000
DIM = 128
BATCH = 4096
HIST = 50

_NC = 2
_NS = 16
_NW = _NC * _NS

_SPAN = BATCH // _NW
_NBUF = 7
_STEADY = (HIST - _NBUF) // _NBUF


def _embed_grid(idx_hbm, table_hbm, out_hbm, idx_v, *bufs):
    rows = bufs[:_NBUF]
    gsem = bufs[_NBUF:2 * _NBUF]
    ssem = bufs[2 * _NBUF:]
    w = lax.axis_index("s") * _NC + lax.axis_index("c")
    col0 = w * _SPAN
    pltpu.sync_copy(idx_hbm.at[pl.ds(0, 8), pl.ds(col0, _SPAN)],
                    idx_v.at[pl.ds(0, 8)])

    def start_gather(b, h):
        pltpu.async_copy(table_hbm.at[idx_v.at[h]], rows[b], gsem[b])

    def wait_gather(b):
        pltpu.make_async_copy(table_hbm.at[pl.ds(0, _SPAN)], rows[b],
                              gsem[b]).wait()

    def start_store(b, h):
        pltpu.async_copy(rows[b], out_hbm.at[h, pl.ds(col0, _SPAN)], ssem[b])

    def wait_store(b):
        pltpu.make_async_copy(rows[b], out_hbm.at[0, pl.ds(col0, _SPAN)],
                              ssem[b]).wait()

    for b in range(_NBUF):
        start_gather(b, b)
    pltpu.sync_copy(idx_hbm.at[pl.ds(8, HIST - 8), pl.ds(col0, _SPAN)],
                    idx_v.at[pl.ds(8, HIST - 8)])

    def outer(t, carry):
        for b in range(_NBUF):
            h = t * _NBUF + b
            wait_gather(b)
            start_store(b, h)
            wait_store(b)
            start_gather(b, h + _NBUF)
        return carry

    lax.fori_loop(0, _STEADY, outer, 0)
    for h in range(_STEADY * _NBUF, HIST):
        b = h % _NBUF
        wait_gather(b)
        start_store(b, h)
        if h + _NBUF < HIST:
            wait_store(b)
            start_gather(b, h + _NBUF)
    for b in range(_NBUF):
        wait_store(b)


@jax.jit
def _embed(idx_t, table):
    mesh = plsc.VectorSubcoreMesh(core_axis_name="c", subcore_axis_name="s")
    k = functools.partial(
        pl.kernel,
        out_type=jax.ShapeDtypeStruct((HIST, BATCH, DIM), jnp.float32),
        mesh=mesh,
        scratch_types=(
            [pltpu.VMEM((HIST, _SPAN), jnp.int32)]
            + [pltpu.VMEM((_SPAN, DIM), jnp.float32) for _ in range(_NBUF)]
            + [pltpu.SemaphoreType.DMA for _ in range(2 * _NBUF)]
        ),
    )(_embed_grid)
    out = k(idx_t, table)
    return jnp.transpose(out, (1, 0, 2))


def kernel(word_vector, weight):
    return _embed(word_vector.T.astype(jnp.int32), weight)


# speedup vs baseline: 1.0031x; 1.0031x over previous
import functools

import jax
import jax.numpy as jnp
from jax import lax
from jax.experimental import pallas as pl
from jax.experimental.pallas import tpu as pltpu
from jax.experimental.pallas import tpu_sc as plsc

VOCAB = 100000
DIM = 128
BATCH = 4096
HIST = 50

_NC = 2
_NS = 16
_NW = _NC * _NS

_SPAN = BATCH // _NW
_NBUF = 7
_STEADY = (HIST - _NBUF) // _NBUF


def _embed_grid(idx_hbm, table_hbm, out_hbm, idx_v, *bufs):
    rows = bufs[:_NBUF]
    gsem = bufs[_NBUF:2 * _NBUF]
    ssem = bufs[2 * _NBUF:]
    w = lax.axis_index("s") * _NC + lax.axis_index("c")
    col0 = w * _SPAN
    pltpu.sync_copy(idx_hbm.at[:, pl.ds(col0, _SPAN)], idx_v)

    def start_gather(b, h):
        pltpu.async_copy(table_hbm.at[idx_v.at[h]], rows[b], gsem[b])

    def wait_gather(b):
        pltpu.make_async_copy(table_hbm.at[pl.ds(0, _SPAN)], rows[b],
                              gsem[b]).wait()

    def start_store(b, h):
        pltpu.async_copy(rows[b], out_hbm.at[h, pl.ds(col0, _SPAN)], ssem[b])

    def wait_store(b):
        pltpu.make_async_copy(rows[b], out_hbm.at[0, pl.ds(col0, _SPAN)],
                              ssem[b]).wait()

    for b in range(_NBUF):
        start_gather(b, b)

    def outer(t, carry):
        for b in range(_NBUF):
            h = t * _NBUF + b
            wait_gather(b)
            start_store(b, h)
            wait_store(b)
            start_gather(b, h + _NBUF)
        return carry

    lax.fori_loop(0, _STEADY, outer, 0)
    for h in range(_STEADY * _NBUF, HIST):
        b = h % _NBUF
        wait_gather(b)
        start_store(b, h)
        if h + _NBUF < HIST:
            wait_store(b)
            start_gather(b, h + _NBUF)
    for b in range(_NBUF):
        wait_store(b)


@jax.jit
def _embed(idx_t, table):
    mesh = plsc.VectorSubcoreMesh(core_axis_name="c", subcore_axis_name="s")
    k = functools.partial(
        pl.kernel,
        out_type=jax.ShapeDtypeStruct((HIST, BATCH, DIM), jnp.float32),
        mesh=mesh,
        scratch_types=(
            [pltpu.VMEM((HIST, _SPAN), jnp.int32)]
            + [pltpu.VMEM((_SPAN, DIM), jnp.float32) for _ in range(_NBUF)]
            + [pltpu.SemaphoreType.DMA for _ in range(2 * _NBUF)]
        ),
    )(_embed_grid)
    out = k(idx_t, table)
    return jnp.transpose(out, (1, 0, 2))


def kernel(word_vector, weight):
    return _embed(word_vector.T.astype(jnp.int32), weight)
